# baseline (speedup 1.0000x reference)
import jax
import jax.numpy as jnp
from jax.experimental import pallas as pl
from jax.experimental.pallas import tpu as pltpu

B, N, D_MODEL, D_FF, E = 4, 8192, 768, 1024, 8

_TN = 2048


def _fused_body(x_pool_ref, x_ffn_ref, wr_ref, br_ref,
                w1_hbm, b1_hbm, w2_hbm, b2_hbm,
                out_ref,
                acc_ref, w1_buf, b1_buf, w2_buf, b2_buf,
                scores_vmem, scores_smem,
                sem_s, sem_w1, sem_b1, sem_w2, sem_b2):
    p = pl.program_id(0)
    n = pl.program_id(1)
    nt = pl.num_programs(1)

    @pl.when(p < B)
    def _pool():
        part = jnp.sum(x_pool_ref[0], axis=0, keepdims=True)

        @pl.when(n == 0)
        def _init():
            acc_ref[...] = part

        @pl.when(n != 0)
        def _accum():
            acc_ref[...] = acc_ref[...] + part

        @pl.when(n == nt - 1)
        def _route():
            pooled = acc_ref[...] * (1.0 / N)
            scores = jax.lax.dot_general(
                pooled, wr_ref[...],
                (((1,), (1,)), ((), ())),
                preferred_element_type=jnp.float32,
            ) + br_ref[...]
            scores_vmem[...] = scores
            cp = pltpu.make_async_copy(scores_vmem, scores_smem, sem_s)
            cp.start()
            cp.wait()

            def _amax(i, carry):
                bv, bi = carry
                v = scores_smem[0, i]
                better = v > bv
                return (jnp.where(better, v, bv),
                        jnp.where(better, i, bi))

            _, e = jax.lax.fori_loop(
                1, E, _amax, (scores_smem[0, 0], jnp.int32(0)))

            slot = jax.lax.rem(p, 2)
            pltpu.make_async_copy(w1_hbm.at[e], w1_buf.at[slot], sem_w1).start()
            pltpu.make_async_copy(b1_hbm.at[e], b1_buf.at[slot], sem_b1).start()
            pltpu.make_async_copy(w2_hbm.at[e], w2_buf.at[slot], sem_w2).start()
            pltpu.make_async_copy(b2_hbm.at[e], b2_buf.at[slot], sem_b2).start()

    @pl.when(p > 0)
    def _ffn():
        slot = jax.lax.rem(p - 1, 2)

        @pl.when(n == 0)
        def _wait_w1():
            pltpu.make_async_copy(w1_hbm.at[0], w1_buf.at[slot], sem_w1).wait()
            pltpu.make_async_copy(b1_hbm.at[0], b1_buf.at[slot], sem_b1).wait()

        x = x_ffn_ref[0]
        h = jnp.dot(x, w1_buf[slot], preferred_element_type=jnp.float32)
        h = jnp.maximum(h + b1_buf[slot], 0.0)

        @pl.when(n == 0)
        def _wait_w2():
            pltpu.make_async_copy(w2_hbm.at[0], w2_buf.at[slot], sem_w2).wait()
            pltpu.make_async_copy(b2_hbm.at[0], b2_buf.at[slot], sem_b2).wait()

        out = jnp.dot(h, w2_buf[slot], preferred_element_type=jnp.float32)
        out_ref[0] = out + b2_buf[slot]


def kernel(x, Wr, br, W1, b1, W2, b2):
    nt = N // _TN

    def pool_idx(p, n):
        last = (p == B)
        return (jnp.where(last, B - 1, p), jnp.where(last, nt - 1, n), 0)

    def ffn_idx(p, n):
        first = (p == 0)
        return (jnp.where(first, 0, p - 1), jnp.where(first, 0, n), 0)

    out = pl.pallas_call(
        _fused_body,
        grid=(B + 1, nt),
        in_specs=[
            pl.BlockSpec((1, _TN, D_MODEL), pool_idx),
            pl.BlockSpec((1, _TN, D_MODEL), ffn_idx),
            pl.BlockSpec((E, D_MODEL), lambda p, n: (0, 0)),
            pl.BlockSpec((1, E), lambda p, n: (0, 0)),
            pl.BlockSpec(memory_space=pltpu.MemorySpace.HBM),
            pl.BlockSpec(memory_space=pltpu.MemorySpace.HBM),
            pl.BlockSpec(memory_space=pltpu.MemorySpace.HBM),
            pl.BlockSpec(memory_space=pltpu.MemorySpace.HBM),
        ],
        out_specs=pl.BlockSpec((1, _TN, D_MODEL), ffn_idx),
        out_shape=jax.ShapeDtypeStruct((B, N, D_MODEL), jnp.float32),
        scratch_shapes=[
            pltpu.VMEM((1, D_MODEL), jnp.float32),
            pltpu.VMEM((2, D_MODEL, D_FF), jnp.float32),
            pltpu.VMEM((2, 1, D_FF), jnp.float32),
            pltpu.VMEM((2, D_FF, D_MODEL), jnp.float32),
            pltpu.VMEM((2, 1, D_MODEL), jnp.float32),
            pltpu.VMEM((1, E), jnp.float32),
            pltpu.SMEM((1, E), jnp.float32),
            pltpu.SemaphoreType.DMA,
            pltpu.SemaphoreType.DMA,
            pltpu.SemaphoreType.DMA,
            pltpu.SemaphoreType.DMA,
            pltpu.SemaphoreType.DMA,
        ],
        compiler_params=pltpu.CompilerParams(
            dimension_semantics=("arbitrary", "arbitrary"),
            vmem_limit_bytes=63 * 1024 * 1024,
        ),
    )(x, x, Wr, br.reshape(1, E),
      W1, b1.reshape(E, 1, D_FF), W2, b2.reshape(E, 1, D_MODEL))
    return out


# speedup vs baseline: 1.2472x; 1.2472x over previous
import jax
import jax.numpy as jnp
from jax.experimental import pallas as pl
from jax.experimental.pallas import tpu as pltpu

B, N, D_MODEL, D_FF, E = 4, 8192, 768, 1024, 8

_TN = 2048


def _fused_body(x_pool_ref, x_ffn_ref, wr_ref, br_ref,
                w1_hbm, b1_hbm, w2_hbm, b2_hbm,
                out_ref,
                acc_ref, w1_buf, b1_buf, w2_buf, b2_buf,
                scores_vmem, scores_smem,
                sem_s, sem_w1, sem_b1, sem_w2, sem_b2):
    p = pl.program_id(0)
    n = pl.program_id(1)
    nt = pl.num_programs(1)

    @pl.when(p < B)
    def _pool():
        part = jnp.sum(x_pool_ref[0], axis=0, keepdims=True)

        @pl.when(n == 0)
        def _init():
            acc_ref[...] = part

        @pl.when(n != 0)
        def _accum():
            acc_ref[...] = acc_ref[...] + part

        @pl.when(n == nt - 1)
        def _route():
            pooled = acc_ref[...] * (1.0 / N)
            scores = jax.lax.dot_general(
                pooled, wr_ref[...],
                (((1,), (1,)), ((), ())),
                preferred_element_type=jnp.float32,
            ) + br_ref[...]
            scores_vmem[...] = scores
            cp = pltpu.make_async_copy(scores_vmem, scores_smem, sem_s)
            cp.start()
            cp.wait()

            def _amax(i, carry):
                bv, bi = carry
                v = scores_smem[0, i]
                better = v > bv
                return (jnp.where(better, v, bv),
                        jnp.where(better, i, bi))

            _, e = jax.lax.fori_loop(
                1, E, _amax, (scores_smem[0, 0], jnp.int32(0)))

            slot = jax.lax.rem(p, 2)
            pltpu.make_async_copy(w1_hbm.at[e], w1_buf.at[slot], sem_w1).start()
            pltpu.make_async_copy(b1_hbm.at[e], b1_buf.at[slot], sem_b1).start()
            pltpu.make_async_copy(w2_hbm.at[e], w2_buf.at[slot], sem_w2).start()
            pltpu.make_async_copy(b2_hbm.at[e], b2_buf.at[slot], sem_b2).start()

    @pl.when(p > 0)
    def _ffn():
        slot = jax.lax.rem(p - 1, 2)

        @pl.when(n == 0)
        def _wait_weights():
            pltpu.make_async_copy(w1_hbm.at[0], w1_buf.at[slot], sem_w1).wait()
            pltpu.make_async_copy(b1_hbm.at[0], b1_buf.at[slot], sem_b1).wait()
            pltpu.make_async_copy(w2_hbm.at[0], w2_buf.at[slot], sem_w2).wait()
            pltpu.make_async_copy(b2_hbm.at[0], b2_buf.at[slot], sem_b2).wait()

        x = x_ffn_ref[0]
        h = jnp.dot(x, w1_buf[slot], preferred_element_type=jnp.float32)
        h = jnp.maximum(h + b1_buf[slot], 0.0)
        out = jnp.dot(h, w2_buf[slot], preferred_element_type=jnp.float32)
        out_ref[0] = out + b2_buf[slot]


def kernel(x, Wr, br, W1, b1, W2, b2):
    nt = N // _TN

    def pool_idx(p, n):
        last = (p == B)
        return (jnp.where(last, B - 1, p), jnp.where(last, nt - 1, n), 0)

    def ffn_idx(p, n):
        first = (p == 0)
        return (jnp.where(first, 0, p - 1), jnp.where(first, 0, n), 0)

    out = pl.pallas_call(
        _fused_body,
        grid=(B + 1, nt),
        in_specs=[
            pl.BlockSpec((1, _TN, D_MODEL), pool_idx),
            pl.BlockSpec((1, _TN, D_MODEL), ffn_idx),
            pl.BlockSpec((E, D_MODEL), lambda p, n: (0, 0)),
            pl.BlockSpec((1, E), lambda p, n: (0, 0)),
            pl.BlockSpec(memory_space=pltpu.MemorySpace.HBM),
            pl.BlockSpec(memory_space=pltpu.MemorySpace.HBM),
            pl.BlockSpec(memory_space=pltpu.MemorySpace.HBM),
            pl.BlockSpec(memory_space=pltpu.MemorySpace.HBM),
        ],
        out_specs=pl.BlockSpec((1, _TN, D_MODEL), ffn_idx),
        out_shape=jax.ShapeDtypeStruct((B, N, D_MODEL), jnp.float32),
        scratch_shapes=[
            pltpu.VMEM((1, D_MODEL), jnp.float32),
            pltpu.VMEM((2, D_MODEL, D_FF), jnp.float32),
            pltpu.VMEM((2, 1, D_FF), jnp.float32),
            pltpu.VMEM((2, D_FF, D_MODEL), jnp.float32),
            pltpu.VMEM((2, 1, D_MODEL), jnp.float32),
            pltpu.VMEM((1, E), jnp.float32),
            pltpu.SMEM((1, E), jnp.float32),
            pltpu.SemaphoreType.DMA,
            pltpu.SemaphoreType.DMA,
            pltpu.SemaphoreType.DMA,
            pltpu.SemaphoreType.DMA,
            pltpu.SemaphoreType.DMA,
        ],
        compiler_params=pltpu.CompilerParams(
            dimension_semantics=("arbitrary", "arbitrary"),
        ),
    )(x, x, Wr, br.reshape(1, E),
      W1, b1.reshape(E, 1, D_FF), W2, b2.reshape(E, 1, D_MODEL))
    return out
